# Initial kernel scaffold; baseline (speedup 1.0000x reference)
import jax
import jax.numpy as jnp
from jax.experimental import pallas as pl


def kernel(dec_output, final_output, attention_weights, encoder_input, W, b, inp_shape, tar_shape, training):
    raise NotImplementedError("write your pallas kernel here")


# speedup vs baseline: 1.3599x; 1.3599x over previous
import functools

import jax
import jax.numpy as jnp
from jax import lax
from jax.experimental import pallas as pl
from jax.experimental.pallas import tpu as pltpu
from jax.experimental.pallas import tpu_sc as plsc


def _stage1_body(H, TAR, INP, VOCAB, JBS,
                 attn_ref, dec_ref, enc_row_ref, enc_col_ref, w_ref, b_ref,
                 lc_ref, gidx_ref):
  bi = pl.program_id(0)
  a = attn_ref[0]
  m = a[0:TAR, :]
  for h in range(1, H):
    m = m + a[h * TAR:(h + 1) * TAR, :]
  m = m * (1.0 / H)
  mmax = jnp.max(m, axis=-1, keepdims=True)
  e = jnp.exp(m - mmax)
  dist = e / jnp.sum(e, axis=-1, keepdims=True)
  x = jnp.dot(dec_ref[0], w_ref[...],
              preferred_element_type=jnp.float32) + b_ref[0, 0]
  pg = jax.nn.sigmoid(x)
  upd = (1.0 - pg) * dist
  enc_row = enc_row_ref[0]
  acc = jnp.zeros((TAR, INP), jnp.float32)
  for jb in range(INP // JBS):
    ej = enc_col_ref[0, pl.ds(jb * JBS, JBS), :]
    mjb = (ej == enc_row).astype(jnp.float32)
    acc = acc + jnp.dot(upd[:, jb * JBS:(jb + 1) * JBS], mjb,
                        preferred_element_type=jnp.float32)
  lc_ref[0] = jnp.log(acc)
  t_iota = lax.broadcasted_iota(jnp.int32, (TAR, INP), 0)
  gidx_ref[0] = (bi * TAR + t_iota) * VOCAB + enc_row


def _stage2_body(fin_ref, dec_ref, w_ref, b_ref, out_ref, lse_ref, lpg_ref):
  x = jnp.dot(dec_ref[...], w_ref[...],
              preferred_element_type=jnp.float32) + b_ref[0, 0]
  lpg = jax.nn.log_sigmoid(x)
  row = fin_ref[...]
  mmax = jnp.max(row, axis=-1, keepdims=True)
  lse = mmax + jnp.log(jnp.sum(jnp.exp(row - mmax), axis=-1, keepdims=True))
  out_ref[...] = row - lse + lpg
  lse_ref[...] = lse
  lpg_ref[...] = lpg


def _stage4_body(fing_ref, lse_ref, lpg_ref, lc_ref, fix_ref):
  g = fing_ref[...] - lse_ref[...] + lpg_ref[...]
  fix_ref[...] = jnp.logaddexp(g, lc_ref[...])


def kernel(dec_output, final_output, attention_weights, encoder_input,
           W, b, inp_shape, tar_shape, training):
  B, TAR, D = dec_output.shape
  VOCAB = final_output.shape[-1]
  H = attention_weights.shape[1]
  INP = encoder_input.shape[1]
  R = B * TAR
  N = R * VOCAB
  JBS = 512

  attn_r = attention_weights.reshape(B, H * TAR, INP)
  enc_row = encoder_input.reshape(B, 1, INP)
  enc_col = encoder_input.reshape(B, INP, 1)
  b2 = b.reshape(1, 1)
  dec2 = dec_output.reshape(R, D)
  fin2 = final_output.reshape(R, VOCAB)

  lc, gidx = pl.pallas_call(
      functools.partial(_stage1_body, H, TAR, INP, VOCAB, JBS),
      grid=(B,),
      in_specs=[
          pl.BlockSpec((1, H * TAR, INP), lambda i: (i, 0, 0)),
          pl.BlockSpec((1, TAR, D), lambda i: (i, 0, 0)),
          pl.BlockSpec((1, 1, INP), lambda i: (i, 0, 0)),
          pl.BlockSpec((1, INP, 1), lambda i: (i, 0, 0)),
          pl.BlockSpec((D, 1), lambda i: (0, 0)),
          pl.BlockSpec((1, 1), lambda i: (0, 0)),
      ],
      out_specs=[
          pl.BlockSpec((1, TAR, INP), lambda i: (i, 0, 0)),
          pl.BlockSpec((1, TAR, INP), lambda i: (i, 0, 0)),
      ],
      out_shape=[
          jax.ShapeDtypeStruct((B, TAR, INP), jnp.float32),
          jax.ShapeDtypeStruct((B, TAR, INP), jnp.int32),
      ],
  )(attn_r, dec_output, enc_row, enc_col, W, b2)

  RB = 8
  dense, lse, lpg = pl.pallas_call(
      _stage2_body,
      grid=(R // RB,),
      in_specs=[
          pl.BlockSpec((RB, VOCAB), lambda i: (i, 0)),
          pl.BlockSpec((RB, D), lambda i: (i, 0)),
          pl.BlockSpec((D, 1), lambda i: (0, 0)),
          pl.BlockSpec((1, 1), lambda i: (0, 0)),
      ],
      out_specs=[
          pl.BlockSpec((RB, VOCAB), lambda i: (i, 0)),
          pl.BlockSpec((RB, 1), lambda i: (i, 0)),
          pl.BlockSpec((RB, 1), lambda i: (i, 0)),
      ],
      out_shape=[
          jax.ShapeDtypeStruct((R, VOCAB), jnp.float32),
          jax.ShapeDtypeStruct((R, 1), jnp.float32),
          jax.ShapeDtypeStruct((R, 1), jnp.float32),
      ],
  )(fin2, dec2, W, b2)

  NC, NS = 2, 16
  NW = NC * NS
  K = R * INP
  CW = 128
  C = K // (NW * CW)
  mesh = plsc.VectorSubcoreMesh(core_axis_name="c", subcore_axis_name="s")
  gidx3 = gidx.reshape(NW, C, CW)
  fin_flat = final_output.reshape(N)

  @functools.partial(
      pl.kernel,
      out_type=jax.ShapeDtypeStruct((NW, C, CW), jnp.float32),
      mesh=mesh,
      scratch_types=[
          pltpu.VMEM((C, CW), jnp.int32),
          pltpu.VMEM((C, CW), jnp.float32),
          pltpu.SemaphoreType.DMA,
      ],
  )
  def sc_gather(fin_hbm, idx_hbm, out_hbm, idx_v, val_v, sem):
    w = lax.axis_index("s") * NC + lax.axis_index("c")
    pltpu.sync_copy(idx_hbm.at[w], idx_v)
    for g in range(C // 8):
      hs = [pltpu.async_copy(fin_hbm.at[idx_v.at[g * 8 + j]],
                             val_v.at[g * 8 + j], sem)
            for j in range(8)]
      for h in hs:
        h.wait()
    pltpu.sync_copy(val_v, out_hbm.at[w])

  fin_g = sc_gather(fin_flat, gidx3)

  fix = pl.pallas_call(
      _stage4_body,
      grid=(1,),
      in_specs=[
          pl.BlockSpec((R, INP), lambda i: (0, 0)),
          pl.BlockSpec((R, 1), lambda i: (0, 0)),
          pl.BlockSpec((R, 1), lambda i: (0, 0)),
          pl.BlockSpec((R, INP), lambda i: (0, 0)),
      ],
      out_specs=pl.BlockSpec((R, INP), lambda i: (0, 0)),
      out_shape=jax.ShapeDtypeStruct((R, INP), jnp.float32),
  )(fin_g.reshape(R, INP), lse, lpg, lc.reshape(R, INP))

  @functools.partial(
      pl.kernel,
      out_type=(),
      mesh=mesh,
      scratch_types=[
          pltpu.VMEM((C, CW), jnp.int32),
          pltpu.VMEM((C, CW), jnp.float32),
          pltpu.SemaphoreType.DMA,
      ],
  )
  def sc_scatter(idx_hbm, fix_hbm, dense_ref, idx_v, val_v, sem):
    w = lax.axis_index("s") * NC + lax.axis_index("c")
    pltpu.sync_copy(idx_hbm.at[w], idx_v)
    pltpu.sync_copy(fix_hbm.at[w], val_v)
    for g in range(C // 8):
      hs = [pltpu.async_copy(val_v.at[g * 8 + j],
                             dense_ref.at[idx_v.at[g * 8 + j]], sem)
            for j in range(8)]
      for h in hs:
        h.wait()

  dense_ref = jax.new_ref(dense.reshape(N))
  sc_scatter(gidx3, fix.reshape(NW, C, CW), dense_ref)
  return dense_ref[...].reshape(B, TAR, VOCAB)
